# baseline (speedup 1.0000x reference)
import dataclasses
import functools

import jax
import jax.numpy as jnp
from jax import lax
from jax.experimental import pallas as pl
from jax.experimental.pallas import tpu as pltpu
from jax.experimental.pallas import tpu_sc as plsc

_N = 10000
_E = 320000
_D = 128
_O = 128
_R = 32
_B = 8

_NC = 2
_NS = 16
_L = 16

_C = 128
_SUP = 8
_SUPE = _SUP * _C
_NSUP = 10
_CH_PER_SUB = _NSUP * _SUP
_EP = _NC * _NS * _CH_PER_SUB * _C

_ROWS_PER_SUB = 624
_ROWS_TAIL = _N - _NS * _ROWS_PER_SUB

_RG = 8


def _tc_weights_body(coeff_ref, bases_ref, out_ref):
    r = pl.program_id(0)
    w = coeff_ref[r, 0] * bases_ref[0]
    for b in range(1, _B):
        w += coeff_ref[r, b] * bases_ref[b]
    out_ref[...] = w


def _tc_weights(bases, coeff):
    return pl.pallas_call(
        _tc_weights_body,
        grid=(_R,),
        in_specs=[
            pl.BlockSpec(memory_space=pltpu.SMEM),
            pl.BlockSpec((_B, _D, _O), lambda r: (0, 0, 0)),
        ],
        out_specs=pl.BlockSpec((_D, _O), lambda r: (0, r)),
        out_shape=jax.ShapeDtypeStruct((_D, _R * _O), jnp.float32),
    )(coeff, bases)


def _tc_transform_body(h_ref, w_ref, out_ref):
    res = jnp.dot(h_ref[...], w_ref[...], preferred_element_type=jnp.float32)
    for k in range(_RG):
        out_ref[k] = res[:, k * _O:(k + 1) * _O]


def _tc_transform(h, wflat):
    nb = 5
    rows = _N // nb
    return pl.pallas_call(
        _tc_transform_body,
        grid=(nb, _R // _RG),
        in_specs=[
            pl.BlockSpec((rows, _D), lambda n, g: (n, 0)),
            pl.BlockSpec((_D, _RG * _O), lambda n, g: (0, g)),
        ],
        out_specs=pl.BlockSpec((_RG, rows, _O), lambda n, g: (g, n, 0)),
        out_shape=jax.ShapeDtypeStruct((_R, _N, _O), jnp.float32),
    )(h, wflat)


def _scale_rows(rows_ref, norm_ref, nbase):
    @plsc.parallel_loop(0, _C, 1, unroll=4)
    def _(e):
        esplat = jnp.full((_L,), nbase + e, jnp.int32)
        nsplat = plsc.load_gather(norm_ref, [esplat])
        for k in range(_O // _L):
            sl = pl.ds(k * _L, _L)
            rows_ref[e, sl] = rows_ref[e, sl] * nsplat


def _sc_edge_kernel_body(allt_hbm, src_hbm, dst_hbm, rel_hbm, norm_hbm,
                         out_hbm,
                         srcA, relA, dst1A, normA, idxA, dst2A,
                         srcB, relB, dst1B, normB, idxB, dst2B,
                         rows0, rows1, acc_sh,
                         dsemA, dsemB, gsem0, gsem1, ssem0, ssem1):
    cid = lax.axis_index("c")
    sid = lax.axis_index("s")
    rows = (rows0, rows1)
    gsem = (gsem0, gsem1)
    ssem = (ssem0, ssem1)
    sets = (
        dict(src=srcA, rel=relA, dst1=dst1A, norm=normA, idx=idxA,
             dst2=dst2A, dsem=dsemA),
        dict(src=srcB, rel=relB, dst1=dst1B, norm=normB, idx=idxB,
             dst2=dst2B, dsem=dsemB),
    )

    zvec = jnp.zeros((_L,), jnp.float32)

    @pl.loop(0, _C)
    def _(i):
        @pl.loop(0, _O, step=_L)
        def _(k):
            rows0[i, pl.ds(k, _L)] = zvec

    @pl.loop(0, _ROWS_PER_SUB - _C + 1, step=_C)
    def _(j):
        pltpu.sync_copy(rows0, acc_sh.at[pl.ds(sid * _ROWS_PER_SUB + j, _C)])

    pltpu.sync_copy(rows0.at[pl.ds(0, 112)],
                    acc_sh.at[pl.ds(sid * _ROWS_PER_SUB + 4 * _C, 112)])

    @pl.when(sid == _NS - 1)
    def _():
        pltpu.sync_copy(rows0.at[pl.ds(0, _ROWS_TAIL)],
                        acc_sh.at[pl.ds(_NS * _ROWS_PER_SUB, _ROWS_TAIL)])

    plsc.subcore_barrier()

    base_ch = (cid * _NS + sid) * _CH_PER_SUB

    def issue_desc(s, st):
        e0 = (base_ch + s * _SUP) * _C
        return [
            pltpu.async_copy(src_hbm.at[pl.ds(e0, _SUPE)], st["src"],
                             st["dsem"]),
            pltpu.async_copy(rel_hbm.at[pl.ds(e0, _SUPE)], st["rel"],
                             st["dsem"]),
            pltpu.async_copy(dst_hbm.at[pl.ds(e0, _SUPE)], st["dst1"],
                             st["dsem"]),
            pltpu.async_copy(norm_hbm.at[pl.ds(e0, _SUPE)], st["norm"],
                             st["dsem"]),
        ]

    def compute_idx(st):
        @pl.loop(0, _SUPE, step=_L)
        def _(i):
            st["idx"][pl.ds(i, _L)] = (st["rel"][pl.ds(i, _L)] * _N
                                       + st["src"][pl.ds(i, _L)])
            j = i // _C
            k = i - j * _C
            st["dst2"][j, pl.ds(k, _L)] = st["dst1"][pl.ds(i, _L)]

    pend_scat = [None, None]
    pend_desc = issue_desc(0, sets[0])
    for s in range(_NSUP):
        cur = sets[s % 2]
        for d in pend_desc:
            d.wait()
        if s + 1 < _NSUP:
            pend_desc = issue_desc(s + 1, sets[(s + 1) % 2])
        compute_idx(cur)

        gathers = [None] * _SUP
        if pend_scat[0] is not None:
            pend_scat[0].wait()
            pend_scat[0] = None
        gathers[0] = pltpu.async_copy(
            allt_hbm.at[cur["idx"].at[pl.ds(0, _C)]], rows0, gsem0)
        for j in range(_SUP):
            p = j % 2
            if j + 1 < _SUP:
                q = 1 - p
                if pend_scat[q] is not None:
                    pend_scat[q].wait()
                    pend_scat[q] = None
                gathers[j + 1] = pltpu.async_copy(
                    allt_hbm.at[cur["idx"].at[pl.ds((j + 1) * _C, _C)]],
                    rows[q], gsem[q])
            gathers[j].wait()
            _scale_rows(rows[p], cur["norm"], j * _C)
            pend_scat[p] = pltpu.async_copy(
                rows[p], acc_sh.at[cur["dst2"].at[j]], ssem[p], add=True)

    pend_scat[0].wait()
    pend_scat[1].wait()

    plsc.subcore_barrier()

    r0 = sid * _ROWS_PER_SUB
    pltpu.sync_copy(acc_sh.at[pl.ds(r0, _ROWS_PER_SUB)],
                    out_hbm.at[cid].at[pl.ds(r0, _ROWS_PER_SUB)])

    @pl.when(sid == _NS - 1)
    def _():
        t0 = _NS * _ROWS_PER_SUB
        pltpu.sync_copy(acc_sh.at[pl.ds(t0, _ROWS_TAIL)],
                        out_hbm.at[cid].at[pl.ds(t0, _ROWS_TAIL)])


def _sc_edges(allt, src, dst, rel, norm_flat):
    mesh = plsc.VectorSubcoreMesh(core_axis_name="c", subcore_axis_name="s")
    cp = pltpu.CompilerParams()
    if "needs_layout_passes" in pltpu.CompilerParams.__dataclass_fields__:
        cp = dataclasses.replace(cp, needs_layout_passes=False)
    desc_set = [
        pltpu.VMEM((_SUPE,), jnp.int32),
        pltpu.VMEM((_SUPE,), jnp.int32),
        pltpu.VMEM((_SUPE,), jnp.int32),
        pltpu.VMEM((_SUPE,), jnp.float32),
        pltpu.VMEM((_SUPE,), jnp.int32),
        pltpu.VMEM((_SUP, _C), jnp.int32),
    ]
    kern = pl.kernel(
        _sc_edge_kernel_body,
        out_type=jax.ShapeDtypeStruct((_NC, _N, _O), jnp.float32),
        mesh=mesh,
        scratch_types=desc_set + desc_set + [
            pltpu.VMEM((_C, _O), jnp.float32),
            pltpu.VMEM((_C, _O), jnp.float32),
            pltpu.VMEM_SHARED((_N, _O), jnp.float32),
            pltpu.SemaphoreType.DMA,
            pltpu.SemaphoreType.DMA,
            pltpu.SemaphoreType.DMA,
            pltpu.SemaphoreType.DMA,
            pltpu.SemaphoreType.DMA,
            pltpu.SemaphoreType.DMA,
        ],
        compiler_params=cp,
    )
    return kern(allt, src, dst, rel, norm_flat)


def _tc_combine_body(p_ref, bias_ref, o_ref):
    o_ref[...] = jnp.maximum(p_ref[0] + p_ref[1] + bias_ref[...], 0.0)


def _tc_combine(parts, bias2d):
    nb = 10
    rows = _N // nb
    return pl.pallas_call(
        _tc_combine_body,
        grid=(nb,),
        in_specs=[
            pl.BlockSpec((_NC, rows, _O), lambda i: (0, i, 0)),
            pl.BlockSpec((1, _O), lambda i: (0, 0)),
        ],
        out_specs=pl.BlockSpec((rows, _O), lambda i: (i, 0)),
        out_shape=jax.ShapeDtypeStruct((_N, _O), jnp.float32),
    )(parts, bias2d)


def kernel(h, edge_index, r, norm, bases, coeff, bias):
    wflat = _tc_weights(bases, coeff)
    allt = _tc_transform(h, wflat).reshape(_R * _N, _O)
    pad = _EP - _E
    spread = jnp.arange(pad, dtype=jnp.int32) % _N
    src = jnp.concatenate([edge_index[0], spread])
    dst = jnp.concatenate([edge_index[1], spread])
    rel = jnp.pad(r, (0, pad))
    nrm = jnp.pad(norm.reshape(_E), (0, pad))
    parts = _sc_edges(allt, src, dst, rel, nrm)
    return _tc_combine(parts, bias.reshape(1, _O))


# speedup vs baseline: 1.0205x; 1.0205x over previous
import dataclasses
import functools

import jax
import jax.numpy as jnp
from jax import lax
from jax.experimental import pallas as pl
from jax.experimental.pallas import tpu as pltpu
from jax.experimental.pallas import tpu_sc as plsc

_N = 10000
_E = 320000
_D = 128
_O = 128
_R = 32
_B = 8

_NC = 2
_NS = 16
_L = 16

_C = 128
_SUP = 8
_SUPE = _SUP * _C
_NSUP = 10
_CH_PER_SUB = _NSUP * _SUP
_EP = _NC * _NS * _CH_PER_SUB * _C

_ROWS_PER_SUB = 624
_ROWS_TAIL = _N - _NS * _ROWS_PER_SUB

_RG = 8


def _tc_weights_body(coeff_ref, bases_ref, out_ref):
    r = pl.program_id(0)
    w = coeff_ref[r, 0] * bases_ref[0]
    for b in range(1, _B):
        w += coeff_ref[r, b] * bases_ref[b]
    out_ref[...] = w


def _tc_weights(bases, coeff):
    return pl.pallas_call(
        _tc_weights_body,
        grid=(_R,),
        in_specs=[
            pl.BlockSpec(memory_space=pltpu.SMEM),
            pl.BlockSpec((_B, _D, _O), lambda r: (0, 0, 0)),
        ],
        out_specs=pl.BlockSpec((_D, _O), lambda r: (0, r)),
        out_shape=jax.ShapeDtypeStruct((_D, _R * _O), jnp.float32),
    )(coeff, bases)


def _tc_transform_body(h_ref, w_ref, out_ref):
    res = jnp.dot(h_ref[...], w_ref[...], preferred_element_type=jnp.float32)
    for k in range(_RG):
        out_ref[k] = res[:, k * _O:(k + 1) * _O]


def _tc_transform(h, wflat):
    nb = 2
    rows = _N // nb
    return pl.pallas_call(
        _tc_transform_body,
        grid=(nb, _R // _RG),
        in_specs=[
            pl.BlockSpec((rows, _D), lambda n, g: (n, 0)),
            pl.BlockSpec((_D, _RG * _O), lambda n, g: (0, g)),
        ],
        out_specs=pl.BlockSpec((_RG, rows, _O), lambda n, g: (g, n, 0)),
        out_shape=jax.ShapeDtypeStruct((_R, _N, _O), jnp.float32),
    )(h, wflat)


def _scale_rows(rows_ref, norm_ref, nbase):
    @plsc.parallel_loop(0, _C, 1, unroll=2)
    def _(e):
        esplat = jnp.full((_L,), nbase + e, jnp.int32)
        nsplat = plsc.load_gather(norm_ref, [esplat])
        for k in range(_O // _L):
            sl = pl.ds(k * _L, _L)
            rows_ref[e, sl] = rows_ref[e, sl] * nsplat


def _sc_edge_kernel_body(allt_hbm, src_hbm, dst_hbm, rel_hbm, norm_hbm,
                         out_hbm,
                         srcA, relA, dst1A, normA, idxA, dst2A,
                         srcB, relB, dst1B, normB, idxB, dst2B,
                         rows0, rows1, acc_sh,
                         dsemA, dsemB, gsem0, gsem1, ssem0, ssem1):
    cid = lax.axis_index("c")
    sid = lax.axis_index("s")
    rows = (rows0, rows1)
    gsem = (gsem0, gsem1)
    ssem = (ssem0, ssem1)
    sets = (
        dict(src=srcA, rel=relA, dst1=dst1A, norm=normA, idx=idxA,
             dst2=dst2A, dsem=dsemA),
        dict(src=srcB, rel=relB, dst1=dst1B, norm=normB, idx=idxB,
             dst2=dst2B, dsem=dsemB),
    )

    zvec = jnp.zeros((_L,), jnp.float32)

    @pl.loop(0, _C)
    def _(i):
        @pl.loop(0, _O, step=_L)
        def _(k):
            rows0[i, pl.ds(k, _L)] = zvec

    @pl.loop(0, _ROWS_PER_SUB - _C + 1, step=_C)
    def _(j):
        pltpu.sync_copy(rows0, acc_sh.at[pl.ds(sid * _ROWS_PER_SUB + j, _C)])

    pltpu.sync_copy(rows0.at[pl.ds(0, 112)],
                    acc_sh.at[pl.ds(sid * _ROWS_PER_SUB + 4 * _C, 112)])

    @pl.when(sid == _NS - 1)
    def _():
        pltpu.sync_copy(rows0.at[pl.ds(0, _ROWS_TAIL)],
                        acc_sh.at[pl.ds(_NS * _ROWS_PER_SUB, _ROWS_TAIL)])

    plsc.subcore_barrier()

    base_ch = (cid * _NS + sid) * _CH_PER_SUB

    def issue_desc(s, st):
        e0 = (base_ch + s * _SUP) * _C
        return [
            pltpu.async_copy(src_hbm.at[pl.ds(e0, _SUPE)], st["src"],
                             st["dsem"]),
            pltpu.async_copy(rel_hbm.at[pl.ds(e0, _SUPE)], st["rel"],
                             st["dsem"]),
            pltpu.async_copy(dst_hbm.at[pl.ds(e0, _SUPE)], st["dst1"],
                             st["dsem"]),
            pltpu.async_copy(norm_hbm.at[pl.ds(e0, _SUPE)], st["norm"],
                             st["dsem"]),
        ]

    def compute_idx(st):
        @pl.loop(0, _SUPE, step=_L)
        def _(i):
            st["idx"][pl.ds(i, _L)] = (st["rel"][pl.ds(i, _L)] * _N
                                       + st["src"][pl.ds(i, _L)])
            j = i // _C
            k = i - j * _C
            st["dst2"][j, pl.ds(k, _L)] = st["dst1"][pl.ds(i, _L)]

    pend_scat = [None, None]
    pend_desc = issue_desc(0, sets[0])
    for s in range(_NSUP):
        cur = sets[s % 2]
        for d in pend_desc:
            d.wait()
        if s + 1 < _NSUP:
            pend_desc = issue_desc(s + 1, sets[(s + 1) % 2])
        compute_idx(cur)

        gathers = [None] * _SUP
        if pend_scat[0] is not None:
            pend_scat[0].wait()
            pend_scat[0] = None
        gathers[0] = pltpu.async_copy(
            allt_hbm.at[cur["idx"].at[pl.ds(0, _C)]], rows0, gsem0)
        for j in range(_SUP):
            p = j % 2
            if j + 1 < _SUP:
                q = 1 - p
                if pend_scat[q] is not None:
                    pend_scat[q].wait()
                    pend_scat[q] = None
                gathers[j + 1] = pltpu.async_copy(
                    allt_hbm.at[cur["idx"].at[pl.ds((j + 1) * _C, _C)]],
                    rows[q], gsem[q])
            gathers[j].wait()
            _scale_rows(rows[p], cur["norm"], j * _C)
            pend_scat[p] = pltpu.async_copy(
                rows[p], acc_sh.at[cur["dst2"].at[j]], ssem[p], add=True)

    pend_scat[0].wait()
    pend_scat[1].wait()

    plsc.subcore_barrier()

    r0 = sid * _ROWS_PER_SUB
    pltpu.sync_copy(acc_sh.at[pl.ds(r0, _ROWS_PER_SUB)],
                    out_hbm.at[cid].at[pl.ds(r0, _ROWS_PER_SUB)])

    @pl.when(sid == _NS - 1)
    def _():
        t0 = _NS * _ROWS_PER_SUB
        pltpu.sync_copy(acc_sh.at[pl.ds(t0, _ROWS_TAIL)],
                        out_hbm.at[cid].at[pl.ds(t0, _ROWS_TAIL)])


def _sc_edges(allt, src, dst, rel, norm_flat):
    mesh = plsc.VectorSubcoreMesh(core_axis_name="c", subcore_axis_name="s")
    cp = pltpu.CompilerParams()
    if "needs_layout_passes" in pltpu.CompilerParams.__dataclass_fields__:
        cp = dataclasses.replace(cp, needs_layout_passes=False)
    desc_set = [
        pltpu.VMEM((_SUPE,), jnp.int32),
        pltpu.VMEM((_SUPE,), jnp.int32),
        pltpu.VMEM((_SUPE,), jnp.int32),
        pltpu.VMEM((_SUPE,), jnp.float32),
        pltpu.VMEM((_SUPE,), jnp.int32),
        pltpu.VMEM((_SUP, _C), jnp.int32),
    ]
    kern = pl.kernel(
        _sc_edge_kernel_body,
        out_type=jax.ShapeDtypeStruct((_NC, _N, _O), jnp.float32),
        mesh=mesh,
        scratch_types=desc_set + desc_set + [
            pltpu.VMEM((_C, _O), jnp.float32),
            pltpu.VMEM((_C, _O), jnp.float32),
            pltpu.VMEM_SHARED((_N, _O), jnp.float32),
            pltpu.SemaphoreType.DMA,
            pltpu.SemaphoreType.DMA,
            pltpu.SemaphoreType.DMA,
            pltpu.SemaphoreType.DMA,
            pltpu.SemaphoreType.DMA,
            pltpu.SemaphoreType.DMA,
        ],
        compiler_params=cp,
    )
    return kern(allt, src, dst, rel, norm_flat)


def _tc_combine_body(p_ref, bias_ref, o_ref):
    o_ref[...] = jnp.maximum(p_ref[0] + p_ref[1] + bias_ref[...], 0.0)


def _tc_combine(parts, bias2d):
    nb = 10
    rows = _N // nb
    return pl.pallas_call(
        _tc_combine_body,
        grid=(nb,),
        in_specs=[
            pl.BlockSpec((_NC, rows, _O), lambda i: (0, i, 0)),
            pl.BlockSpec((1, _O), lambda i: (0, 0)),
        ],
        out_specs=pl.BlockSpec((rows, _O), lambda i: (i, 0)),
        out_shape=jax.ShapeDtypeStruct((_N, _O), jnp.float32),
    )(parts, bias2d)


def kernel(h, edge_index, r, norm, bases, coeff, bias):
    wflat = _tc_weights(bases, coeff)
    allt = _tc_transform(h, wflat).reshape(_R * _N, _O)
    pad = _EP - _E
    spread = jnp.arange(pad, dtype=jnp.int32) % _N
    src = jnp.concatenate([edge_index[0], spread])
    dst = jnp.concatenate([edge_index[1], spread])
    rel = jnp.pad(r, (0, pad))
    nrm = jnp.pad(norm.reshape(_E), (0, pad))
    parts = _sc_edges(allt, src, dst, rel, nrm)
    return _tc_combine(parts, bias.reshape(1, _O))
